# baseline (speedup 1.0000x reference)
import functools

import jax
import jax.numpy as jnp
from jax import lax
from jax.experimental import pallas as pl
from jax.experimental.pallas import tpu as pltpu
from jax.experimental.pallas import tpu_sc as plsc

N = 10000
E = 320000
D_IN = 128
HID = 8
HEADS = 8
N_CLASSES = 2
N_GRAPHS = 64

NC = 2
NS = 16
NW = NC * NS
K = 128
NP = 10240
CHUNKS = 80
EP = NW * K * CHUNKS
RT = NP // NS

_f32 = jnp.float32
_i32 = jnp.int32


def _p1_body(x_ref, w1_ref, am_ref, bm_ref, ht_ref, cb_ref):
    h1 = jnp.dot(x_ref[...], w1_ref[...], preferred_element_type=_f32)
    ht_ref[:, 0:16] = jnp.dot(h1, am_ref[...], preferred_element_type=_f32)
    ht_ref[:, 16:80] = h1
    cb_ref[...] = jnp.dot(h1, bm_ref[...], preferred_element_type=_f32)


def _run_pipeline(issue_idx, wait_idx, issue_gather, wait_gather,
                  compute, issue_scatter, wait_scatter):
    assert CHUNKS == 80
    for j in range(5):
        issue_idx(j, j % 8)
    for j in range(2):
        wait_idx(j % 8)
        issue_gather(j % 8, j % 4)

    def sub(i, k, first=False, do_ef=True, do_g=True):
        bd, bi = k % 4, k % 8
        wait_gather(bi, bd)
        compute(bd)
        issue_scatter(bi, bd)
        if not first:
            wait_scatter((k + 2) % 4)
        if do_ef:
            wait_idx((k + 2) % 8)
            issue_gather((k + 2) % 8, (k + 2) % 4)
        if do_g:
            issue_idx(i + 5, (k + 5) % 8)

    sub(0, 0, first=True)
    sub(1, 1, first=True)

    @pl.loop(0, 9)
    def _outer(i8):
        ib = 2 + 8 * i8
        for k in range(8):
            sub(ib + k, 2 + k)

    for k in range(74, 80):
        sub(k, k, do_ef=(k <= 77), do_g=(k <= 74))
    wait_scatter(2)
    wait_scatter(3)


def _l1_body(src_hbm, dst_hbm, ht_hbm, cb_hbm, z80_hbm,
             acc_out,
             srcv, dstv, hx, brows, acc_s,
             isem, gsem, ssem):
    c = lax.axis_index("c")
    s = lax.axis_index("s")
    wid = s * NC + c
    rows0 = s * RT
    pltpu.sync_copy(z80_hbm.at[pl.ds(rows0, RT)], acc_s.at[pl.ds(rows0, RT)])
    plsc.subcore_barrier()

    lane = lax.broadcasted_iota(_i32, (16,), 0)
    qsel = lane >> 3
    base = wid * (EP // NW)

    def issue_idx(i, bi):
        off = base + i * K
        pltpu.async_copy(src_hbm.at[pl.ds(off, K)], srcv.at[bi], isem.at[bi])
        pltpu.async_copy(dst_hbm.at[pl.ds(off, K)], dstv.at[bi], isem.at[bi])

    def wait_idx(bi):
        pltpu.make_async_copy(src_hbm.at[pl.ds(0, K)], srcv.at[bi], isem.at[bi]).wait()
        pltpu.make_async_copy(dst_hbm.at[pl.ds(0, K)], dstv.at[bi], isem.at[bi]).wait()

    def issue_gather(bi, bd):
        pltpu.async_copy(ht_hbm.at[srcv.at[bi]], hx.at[bd], gsem.at[bd])
        pltpu.async_copy(cb_hbm.at[dstv.at[bi]], brows.at[bd], gsem.at[bd])

    def wait_gather(bi, bd):
        pltpu.make_async_copy(ht_hbm.at[pl.ds(0, K)], hx.at[bd], gsem.at[bd]).wait()
        pltpu.make_async_copy(cb_hbm.at[pl.ds(0, K)], brows.at[bd], gsem.at[bd]).wait()

    def issue_scatter(bi, bd):
        pltpu.async_copy(hx.at[bd], acc_s.at[dstv.at[bi]], ssem.at[bd], add=True)

    def wait_scatter(bd):
        pltpu.make_async_copy(hx.at[bd], acc_s.at[pl.ds(0, K)], ssem.at[bd]).wait()

    def compute(bd):
        hxb = hx.at[bd]
        brb = brows.at[bd]

        @pl.loop(0, K, unroll=2)
        def _edge(e):
            t = hxb[e, 0] + brb[e]
            t = jnp.where(t >= 0, t, 0.2 * t)
            w = jnp.exp(t)
            hxb[e, 0] = w
            for q in range(4):
                wq = w.at[qsel + 2 * q].get(mode="promise_in_bounds")
                hxb[e, 1 + q] = hxb[e, 1 + q] * wq

    _run_pipeline(issue_idx, wait_idx, issue_gather, wait_gather,
                  compute, issue_scatter, wait_scatter)

    plsc.subcore_barrier()
    pltpu.sync_copy(acc_s.at[pl.ds(rows0, RT)], acc_out.at[c, pl.ds(rows0, RT)])


def _p3_body(ap_ref, r16_ref, b1_ref, w2_ref,
             ma_ref, mb_ref, c3_ref, ta_ref, tb_ref):
    acc = ap_ref[0:N, 16:80] + ap_ref[NP:NP + N, 16:80]
    den = ap_ref[0:N, 0:16] + ap_ref[NP:NP + N, 0:16]
    deno = jnp.dot(den, r16_ref[...], preferred_element_type=_f32)
    h = acc / (deno + 1e-16) + b1_ref[...]
    h = jnp.where(h > 0, h, jnp.exp(h) - 1.0)
    h2 = jnp.dot(h, w2_ref[...], preferred_element_type=_f32)
    ta_ref[...] = jnp.dot(h2, ma_ref[...], preferred_element_type=_f32) + c3_ref[...]
    tb_ref[...] = jnp.dot(h2, mb_ref[...], preferred_element_type=_f32)


def _l2_body(src_hbm, dst_hbm, ta_hbm, tb_hbm, z16_hbm,
             acc_out,
             srcv, dstv, arows, brows, rowbuf, acc_s,
             isem, gsem, ssem):
    c = lax.axis_index("c")
    s = lax.axis_index("s")
    wid = s * NC + c
    rows0 = s * RT
    pltpu.sync_copy(z16_hbm.at[pl.ds(rows0, RT)], acc_s.at[pl.ds(rows0, RT)])
    plsc.subcore_barrier()

    lane = lax.broadcasted_iota(_i32, (16,), 0)
    zero16 = jnp.zeros((16,), _i32)
    pat = jnp.where(lane == 0, 3, jnp.where(lane < 3, lane, 4))
    base = wid * (EP // NW)

    def issue_idx(i, bi):
        off = base + i * K
        pltpu.async_copy(src_hbm.at[pl.ds(off, K)], srcv.at[bi], isem.at[bi])
        pltpu.async_copy(dst_hbm.at[pl.ds(off, K)], dstv.at[bi], isem.at[bi])

    def wait_idx(bi):
        pltpu.make_async_copy(src_hbm.at[pl.ds(0, K)], srcv.at[bi], isem.at[bi]).wait()
        pltpu.make_async_copy(dst_hbm.at[pl.ds(0, K)], dstv.at[bi], isem.at[bi]).wait()

    def issue_gather(bi, bd):
        pltpu.async_copy(ta_hbm.at[srcv.at[bi]], arows.at[bd], gsem.at[bd])
        pltpu.async_copy(tb_hbm.at[dstv.at[bi]], brows.at[bd], gsem.at[bd])

    def wait_gather(bi, bd):
        pltpu.make_async_copy(ta_hbm.at[pl.ds(0, K)], arows.at[bd], gsem.at[bd]).wait()
        pltpu.make_async_copy(tb_hbm.at[pl.ds(0, K)], brows.at[bd], gsem.at[bd]).wait()

    def issue_scatter(bi, bd):
        pltpu.async_copy(rowbuf.at[bd], acc_s.at[dstv.at[bi]], ssem.at[bd], add=True)

    def wait_scatter(bd):
        pltpu.make_async_copy(rowbuf.at[bd], acc_s.at[pl.ds(0, K)], ssem.at[bd]).wait()

    def compute(bd):
        arb = arows.at[bd]
        brb = brows.at[bd]
        rwb = rowbuf.at[bd]

        @pl.loop(0, K, unroll=2)
        def _edge(e):
            a = arb[e]
            t = a + brb[e]
            g0 = t.at[zero16].get(mode="promise_in_bounds")
            g0 = jnp.where(g0 >= 0, g0, 0.2 * g0)
            w = jnp.exp(g0)
            mult = a.at[pat].get(mode="promise_in_bounds")
            rwb[e] = w * mult

    _run_pipeline(issue_idx, wait_idx, issue_gather, wait_gather,
                  compute, issue_scatter, wait_scatter)

    plsc.subcore_barrier()
    pltpu.sync_copy(acc_s.at[pl.ds(rows0, RT)], acc_out.at[c, pl.ds(rows0, RT)])


def _p5_body(a2_ref, batch_ref, b2_ref, out_ref):
    acc = a2_ref[0:N] + a2_ref[NP:NP + N]
    den = acc[:, 0:1]
    o = acc[:, 1:3] / (den + 1e-16) + b2_ref[...]
    ids = batch_ref[...]
    g = lax.broadcasted_iota(_i32, (N_GRAPHS, 1), 0)
    mt = (g == ids).astype(_f32)
    sums = jnp.dot(mt, o, preferred_element_type=_f32)
    cnts = jnp.sum(mt, axis=1, keepdims=True)
    pooled = sums / jnp.maximum(cnts, 1.0)
    m = jnp.max(pooled, axis=1, keepdims=True)
    z = pooled - m
    out_ref[...] = z - jnp.log(jnp.sum(jnp.exp(z), axis=1, keepdims=True))


def _sds(shape, dtype=_f32):
    return jax.ShapeDtypeStruct(shape, dtype)


@functools.lru_cache(maxsize=None)
def _sc_kernels():
    mesh = plsc.VectorSubcoreMesh(core_axis_name="c", subcore_axis_name="s")
    params = pltpu.CompilerParams(use_tc_tiling_on_sc=False)
    l1 = pl.kernel(
        _l1_body,
        out_type=_sds((NC, NP, 5, 16)),
        mesh=mesh,
        scratch_types=[
            pltpu.VMEM((8, K), _i32), pltpu.VMEM((8, K), _i32),
            pltpu.VMEM((4, K, 5, 16), _f32), pltpu.VMEM((4, K, 16), _f32),
            pltpu.VMEM_SHARED((NP, 5, 16), _f32),
            pltpu.SemaphoreType.DMA((8,)), pltpu.SemaphoreType.DMA((4,)),
            pltpu.SemaphoreType.DMA((4,)),
        ],
        compiler_params=params,
    )
    l2 = pl.kernel(
        _l2_body,
        out_type=_sds((NC, NP, 16)),
        mesh=mesh,
        scratch_types=[
            pltpu.VMEM((8, K), _i32), pltpu.VMEM((8, K), _i32),
            pltpu.VMEM((4, K, 16), _f32), pltpu.VMEM((4, K, 16), _f32),
            pltpu.VMEM((4, K, 16), _f32),
            pltpu.VMEM_SHARED((NP, 16), _f32),
            pltpu.SemaphoreType.DMA((8,)), pltpu.SemaphoreType.DMA((4,)),
            pltpu.SemaphoreType.DMA((4,)),
        ],
        compiler_params=params,
    )
    return l1, l2


def kernel(x, edge_index, batch, W1, a1_src, a1_dst, b1, W2, a2_src, a2_dst, b2):
    x = x.astype(_f32)
    src = edge_index[0].astype(_i32)
    dst = edge_index[1].astype(_i32)
    pad = EP - E
    src_p = jnp.concatenate([src, jnp.zeros((pad,), _i32)])
    dst_p = jnp.concatenate([dst, jnp.full((pad,), N, _i32)])

    hc = jnp.arange(HEADS * HID)
    hh = jnp.repeat(jnp.arange(HEADS), HID)
    am = jnp.zeros((HEADS * HID, 16), _f32).at[hc, hh].set(a1_src.reshape(-1))
    bm = jnp.zeros((HEADS * HID, 16), _f32).at[hc, hh].set(a1_dst.reshape(-1))
    r16 = jnp.zeros((16, HEADS * HID), _f32).at[hh, hc].set(1.0)
    ma = jnp.zeros((N_CLASSES, 16), _f32)
    ma = ma.at[:, 0].set(a2_src[0]).at[0, 1].set(1.0).at[1, 2].set(1.0)
    mb = jnp.zeros((N_CLASSES, 16), _f32).at[:, 0].set(a2_dst[0])
    c3 = jnp.zeros((1, 16), _f32).at[0, 3].set(1.0)

    htab, cb = pl.pallas_call(
        _p1_body,
        out_shape=[_sds((N, 80)), _sds((N, 16))],
    )(x, W1.astype(_f32), am, bm)

    _l1_kernel, _l2_kernel = _sc_kernels()
    z80 = jnp.zeros((NP, 5, 16), _f32)
    z16 = jnp.zeros((NP, 16), _f32)
    accp = _l1_kernel(src_p, dst_p, htab.reshape(N, 5, 16), cb, z80)

    ta, tb = pl.pallas_call(
        _p3_body,
        out_shape=[_sds((N, 16)), _sds((N, 16))],
    )(accp.reshape(NC * NP, 80), r16, b1.reshape(1, -1).astype(_f32),
      W2.astype(_f32), ma, mb, c3)

    acc2 = _l2_kernel(src_p, dst_p, ta, tb, z16)

    out = pl.pallas_call(
        _p5_body,
        out_shape=_sds((N_GRAPHS, N_CLASSES)),
    )(acc2.reshape(NC * NP, 16), batch.astype(_i32).reshape(1, N),
      b2.reshape(1, -1).astype(_f32))
    return out


# speedup vs baseline: 1.1682x; 1.1682x over previous
import functools

import jax
import jax.numpy as jnp
from jax import lax
from jax.experimental import pallas as pl
from jax.experimental.pallas import tpu as pltpu
from jax.experimental.pallas import tpu_sc as plsc

N = 10000
E = 320000
D_IN = 128
HID = 8
HEADS = 8
N_CLASSES = 2
N_GRAPHS = 64

NC = 2
NS = 16
NW = NC * NS
K = 128
NB = 6
NP = 10240
CHUNKS = 80
EP = NW * K * CHUNKS
RT = NP // NS

_f32 = jnp.float32
_i32 = jnp.int32


def _p1_body(x_ref, w1_ref, am_ref, bm_ref, ht_ref, cb_ref):
    h1 = jnp.dot(x_ref[...], w1_ref[...], preferred_element_type=_f32)
    ht_ref[:, 0:16] = jnp.dot(h1, am_ref[...], preferred_element_type=_f32)
    ht_ref[:, 16:80] = h1
    cb_ref[...] = jnp.dot(h1, bm_ref[...], preferred_element_type=_f32)


def _run_pipeline(issue_idx, wait_idx, issue_gather, wait_gather,
                  compute, issue_scatter, wait_scatter):
    assert CHUNKS == 80
    for j in range(5):
        issue_idx(j, j % 8)
    for j in range(3):
        wait_idx(j % 8)
        issue_gather(j % 8, j % NB)

    def sub(i, k, first=False, do_ef=True, do_g=True):
        bd, bi = k % NB, k % 8
        wait_gather(bi, bd)
        compute(bd)
        issue_scatter(bi, bd)
        if not first:
            wait_scatter((k + 3) % NB)
        if do_ef:
            wait_idx((k + 3) % 8)
            issue_gather((k + 3) % 8, (k + 3) % NB)
        if do_g:
            issue_idx(i + 5, (k + 5) % 8)

    sub(0, 0, first=True)
    sub(1, 1, first=True)
    sub(2, 2, first=True)

    @pl.loop(0, 3)
    def _outer(i24):
        ib = 3 + 24 * i24
        for m in range(24):
            sub(ib + m, 3 + m)

    for k in range(75, 80):
        sub(k, k, do_ef=(k <= 76), do_g=False)
    wait_scatter(77 % NB)
    wait_scatter(78 % NB)
    wait_scatter(79 % NB)


def _l1_body(src_hbm, dst_hbm, ht_hbm, cb_hbm, z80_hbm,
             acc_out,
             srcv, dstv, hx, brows, acc_s,
             isem, gsem, ssem):
    c = lax.axis_index("c")
    s = lax.axis_index("s")
    wid = s * NC + c
    rows0 = s * RT
    pltpu.sync_copy(z80_hbm.at[pl.ds(rows0, RT)], acc_s.at[pl.ds(rows0, RT)])
    plsc.subcore_barrier()

    lane = lax.broadcasted_iota(_i32, (16,), 0)
    qsel = lane >> 3
    base = wid * (EP // NW)

    def issue_idx(i, bi):
        off = base + i * K
        pltpu.async_copy(src_hbm.at[pl.ds(off, K)], srcv.at[bi], isem.at[bi])
        pltpu.async_copy(dst_hbm.at[pl.ds(off, K)], dstv.at[bi], isem.at[bi])

    def wait_idx(bi):
        pltpu.make_async_copy(src_hbm.at[pl.ds(0, K)], srcv.at[bi], isem.at[bi]).wait()
        pltpu.make_async_copy(dst_hbm.at[pl.ds(0, K)], dstv.at[bi], isem.at[bi]).wait()

    def issue_gather(bi, bd):
        pltpu.async_copy(ht_hbm.at[srcv.at[bi]], hx.at[bd], gsem.at[bd])
        pltpu.async_copy(cb_hbm.at[dstv.at[bi]], brows.at[bd], gsem.at[bd])

    def wait_gather(bi, bd):
        pltpu.make_async_copy(ht_hbm.at[pl.ds(0, K)], hx.at[bd], gsem.at[bd]).wait()
        pltpu.make_async_copy(cb_hbm.at[pl.ds(0, K)], brows.at[bd], gsem.at[bd]).wait()

    def issue_scatter(bi, bd):
        pltpu.async_copy(hx.at[bd], acc_s.at[dstv.at[bi]], ssem.at[bd], add=True)

    def wait_scatter(bd):
        pltpu.make_async_copy(hx.at[bd], acc_s.at[pl.ds(0, K)], ssem.at[bd]).wait()

    def compute(bd):
        hxb = hx.at[bd]
        brb = brows.at[bd]

        @pl.loop(0, K)
        def _edge(e):
            t = hxb[e, 0] + brb[e]
            t = jnp.where(t >= 0, t, 0.2 * t)
            w = jnp.exp(t)
            hxb[e, 0] = w
            for q in range(4):
                wq = w.at[qsel + 2 * q].get(mode="promise_in_bounds")
                hxb[e, 1 + q] = hxb[e, 1 + q] * wq

    _run_pipeline(issue_idx, wait_idx, issue_gather, wait_gather,
                  compute, issue_scatter, wait_scatter)

    plsc.subcore_barrier()
    pltpu.sync_copy(acc_s.at[pl.ds(rows0, RT)], acc_out.at[c, pl.ds(rows0, RT)])


def _p3_body(ap_ref, r16_ref, b1_ref, w2_ref,
             ma_ref, mb_ref, c3_ref, ta_ref, tb_ref):
    acc = ap_ref[0:N, 16:80] + ap_ref[NP:NP + N, 16:80]
    den = ap_ref[0:N, 0:16] + ap_ref[NP:NP + N, 0:16]
    deno = jnp.dot(den, r16_ref[...], preferred_element_type=_f32)
    h = acc / (deno + 1e-16) + b1_ref[...]
    h = jnp.where(h > 0, h, jnp.exp(h) - 1.0)
    h2 = jnp.dot(h, w2_ref[...], preferred_element_type=_f32)
    ta_ref[...] = jnp.dot(h2, ma_ref[...], preferred_element_type=_f32) + c3_ref[...]
    tb_ref[...] = jnp.dot(h2, mb_ref[...], preferred_element_type=_f32)


def _l2_body(src_hbm, dst_hbm, ta_hbm, tb_hbm, z16_hbm,
             acc_out,
             srcv, dstv, arows, brows, rowbuf, acc_s,
             isem, gsem, ssem):
    c = lax.axis_index("c")
    s = lax.axis_index("s")
    wid = s * NC + c
    rows0 = s * RT
    pltpu.sync_copy(z16_hbm.at[pl.ds(rows0, RT)], acc_s.at[pl.ds(rows0, RT)])
    plsc.subcore_barrier()

    lane = lax.broadcasted_iota(_i32, (16,), 0)
    zero16 = jnp.zeros((16,), _i32)
    pat = jnp.where(lane == 0, 3, jnp.where(lane < 3, lane, 4))
    base = wid * (EP // NW)

    def issue_idx(i, bi):
        off = base + i * K
        pltpu.async_copy(src_hbm.at[pl.ds(off, K)], srcv.at[bi], isem.at[bi])
        pltpu.async_copy(dst_hbm.at[pl.ds(off, K)], dstv.at[bi], isem.at[bi])

    def wait_idx(bi):
        pltpu.make_async_copy(src_hbm.at[pl.ds(0, K)], srcv.at[bi], isem.at[bi]).wait()
        pltpu.make_async_copy(dst_hbm.at[pl.ds(0, K)], dstv.at[bi], isem.at[bi]).wait()

    def issue_gather(bi, bd):
        pltpu.async_copy(ta_hbm.at[srcv.at[bi]], arows.at[bd], gsem.at[bd])
        pltpu.async_copy(tb_hbm.at[dstv.at[bi]], brows.at[bd], gsem.at[bd])

    def wait_gather(bi, bd):
        pltpu.make_async_copy(ta_hbm.at[pl.ds(0, K)], arows.at[bd], gsem.at[bd]).wait()
        pltpu.make_async_copy(tb_hbm.at[pl.ds(0, K)], brows.at[bd], gsem.at[bd]).wait()

    def issue_scatter(bi, bd):
        pltpu.async_copy(rowbuf.at[bd], acc_s.at[dstv.at[bi]], ssem.at[bd], add=True)

    def wait_scatter(bd):
        pltpu.make_async_copy(rowbuf.at[bd], acc_s.at[pl.ds(0, K)], ssem.at[bd]).wait()

    def compute(bd):
        arb = arows.at[bd]
        brb = brows.at[bd]
        rwb = rowbuf.at[bd]

        @pl.loop(0, K)
        def _edge(e):
            a = arb[e]
            t = a + brb[e]
            g0 = t.at[zero16].get(mode="promise_in_bounds")
            g0 = jnp.where(g0 >= 0, g0, 0.2 * g0)
            w = jnp.exp(g0)
            mult = a.at[pat].get(mode="promise_in_bounds")
            rwb[e] = w * mult

    _run_pipeline(issue_idx, wait_idx, issue_gather, wait_gather,
                  compute, issue_scatter, wait_scatter)

    plsc.subcore_barrier()
    pltpu.sync_copy(acc_s.at[pl.ds(rows0, RT)], acc_out.at[c, pl.ds(rows0, RT)])


def _p5_body(a2_ref, batch_ref, b2_ref, out_ref):
    acc = a2_ref[0:N] + a2_ref[NP:NP + N]
    den = acc[:, 0:1]
    o = acc[:, 1:3] / (den + 1e-16) + b2_ref[...]
    ids = batch_ref[...]
    g = lax.broadcasted_iota(_i32, (N_GRAPHS, 1), 0)
    mt = (g == ids).astype(_f32)
    sums = jnp.dot(mt, o, preferred_element_type=_f32)
    cnts = jnp.sum(mt, axis=1, keepdims=True)
    pooled = sums / jnp.maximum(cnts, 1.0)
    m = jnp.max(pooled, axis=1, keepdims=True)
    z = pooled - m
    out_ref[...] = z - jnp.log(jnp.sum(jnp.exp(z), axis=1, keepdims=True))


def _sds(shape, dtype=_f32):
    return jax.ShapeDtypeStruct(shape, dtype)


@functools.lru_cache(maxsize=None)
def _sc_kernels():
    mesh = plsc.VectorSubcoreMesh(core_axis_name="c", subcore_axis_name="s")
    params = pltpu.CompilerParams(use_tc_tiling_on_sc=False)
    l1 = pl.kernel(
        _l1_body,
        out_type=_sds((NC, NP, 5, 16)),
        mesh=mesh,
        scratch_types=[
            pltpu.VMEM((8, K), _i32), pltpu.VMEM((8, K), _i32),
            pltpu.VMEM((NB, K, 5, 16), _f32), pltpu.VMEM((NB, K, 16), _f32),
            pltpu.VMEM_SHARED((NP, 5, 16), _f32),
            pltpu.SemaphoreType.DMA((8,)), pltpu.SemaphoreType.DMA((NB,)),
            pltpu.SemaphoreType.DMA((NB,)),
        ],
        compiler_params=params,
    )
    l2 = pl.kernel(
        _l2_body,
        out_type=_sds((NC, NP, 16)),
        mesh=mesh,
        scratch_types=[
            pltpu.VMEM((8, K), _i32), pltpu.VMEM((8, K), _i32),
            pltpu.VMEM((NB, K, 16), _f32), pltpu.VMEM((NB, K, 16), _f32),
            pltpu.VMEM((NB, K, 16), _f32),
            pltpu.VMEM_SHARED((NP, 16), _f32),
            pltpu.SemaphoreType.DMA((8,)), pltpu.SemaphoreType.DMA((NB,)),
            pltpu.SemaphoreType.DMA((NB,)),
        ],
        compiler_params=params,
    )
    return l1, l2


def kernel(x, edge_index, batch, W1, a1_src, a1_dst, b1, W2, a2_src, a2_dst, b2):
    x = x.astype(_f32)
    src = edge_index[0].astype(_i32)
    dst = edge_index[1].astype(_i32)
    pad = EP - E
    src_p = jnp.concatenate([src, jnp.zeros((pad,), _i32)])
    dst_p = jnp.concatenate([dst, jnp.full((pad,), N, _i32)])

    hc = jnp.arange(HEADS * HID)
    hh = jnp.repeat(jnp.arange(HEADS), HID)
    am = jnp.zeros((HEADS * HID, 16), _f32).at[hc, hh].set(a1_src.reshape(-1))
    bm = jnp.zeros((HEADS * HID, 16), _f32).at[hc, hh].set(a1_dst.reshape(-1))
    r16 = jnp.zeros((16, HEADS * HID), _f32).at[hh, hc].set(1.0)
    ma = jnp.zeros((N_CLASSES, 16), _f32)
    ma = ma.at[:, 0].set(a2_src[0]).at[0, 1].set(1.0).at[1, 2].set(1.0)
    mb = jnp.zeros((N_CLASSES, 16), _f32).at[:, 0].set(a2_dst[0])
    c3 = jnp.zeros((1, 16), _f32).at[0, 3].set(1.0)

    htab, cb = pl.pallas_call(
        _p1_body,
        out_shape=[_sds((N, 80)), _sds((N, 16))],
    )(x, W1.astype(_f32), am, bm)

    _l1_kernel, _l2_kernel = _sc_kernels()
    z80 = jnp.zeros((NP, 5, 16), _f32)
    z16 = jnp.zeros((NP, 16), _f32)
    accp = _l1_kernel(src_p, dst_p, htab.reshape(N, 5, 16), cb, z80)

    ta, tb = pl.pallas_call(
        _p3_body,
        out_shape=[_sds((N, 16)), _sds((N, 16))],
    )(accp.reshape(NC * NP, 80), r16, b1.reshape(1, -1).astype(_f32),
      W2.astype(_f32), ma, mb, c3)

    acc2 = _l2_kernel(src_p, dst_p, ta, tb, z16)

    out = pl.pallas_call(
        _p5_body,
        out_shape=_sds((N_GRAPHS, N_CLASSES)),
    )(acc2.reshape(NC * NP, 16), batch.astype(_i32).reshape(1, N),
      b2.reshape(1, -1).astype(_f32))
    return out
